# baseline (speedup 1.0000x reference)
import functools

import numpy as np
import jax
import jax.numpy as jnp
from jax import lax
from jax.experimental import pallas as pl
from jax.experimental.pallas import tpu as pltpu
from jax.experimental.pallas import tpu_sc as plsc

_NC = 2
_NS = 16
_NW = _NC * _NS
_LANES = 16
_BLK = 128
_NBUF = 8


def _pos_encoding_np(length: int, d_model: int) -> np.ndarray:
    pos = np.arange(length)[:, np.newaxis].astype(np.float32)
    i = np.arange(d_model)[np.newaxis, :].astype(np.float32)
    angle_rates = 1.0 / np.power(
        10000.0, 2.0 * np.floor(i / 2.0) / np.float32(d_model))
    a = pos * angle_rates
    a[:, 0::2] = np.sin(a[:, 0::2])
    a[:, 1::2] = np.cos(a[:, 1::2])
    return a.astype(np.float32)


@functools.cache
def _make_sc_kernel(B: int, D: int):
    assert B % _BLK == 0 and D % _LANES == 0
    b_per_w = _BLK
    n_active = B // b_per_w
    assert n_active <= _NW
    n_groups = b_per_w // _LANES
    mesh = plsc.VectorSubcoreMesh(
        core_axis_name="c", subcore_axis_name="s",
        num_cores=_NC, num_subcores=_NS)
    def _worker_body(wid, tab_t, idx_hbm, pos_t, out_t,
                     idx_v, acc_v, bufs, gsem, psem):
        base = wid * b_per_w
        pltpu.sync_copy(idx_hbm.at[pl.ds(base, b_per_w)], idx_v)
        pos_cp = pltpu.async_copy(
            pos_t.at[:, pl.ds(base, b_per_w)], acc_v, psem)

        def fire(ids16, l, slot):
            blk = ids16[l] >> 7
            off = pl.multiple_of(blk * _BLK, _BLK)
            pltpu.make_async_copy(
                tab_t.at[:, pl.ds(off, _BLK)],
                bufs.at[slot], gsem.at[slot]).start()

        def extract(m16, l, slot, col):
            pltpu.make_async_copy(
                tab_t.at[:, pl.ds(0, _BLK)],
                bufs.at[slot], gsem.at[slot]).wait()
            m = jnp.full((_LANES,), m16[l], dtype=jnp.int32)
            s = jnp.full((_LANES,), slot, dtype=jnp.int32)
            c = jnp.full((_LANES,), col, dtype=jnp.int32)
            lanes = lax.iota(jnp.int32, _LANES)
            for jc in range(D // _LANES):
                rows = lanes + jc * _LANES
                vals = plsc.load_gather(bufs, [s, rows, m])
                plsc.addupdate_scatter(acc_v, [rows, c], vals)

        ids0 = idx_v[pl.ds(0, _LANES)]
        for l in range(_NBUF):
            fire(ids0, l, l)

        pos_cp.wait()

        def ring_slot(s0, l):
            if _LANES % _NBUF == 0:
                return l % _NBUF
            s = s0 + l
            s = jnp.where(s >= _NBUF, s - _NBUF, s)
            return jnp.where(s >= _NBUF, s - _NBUF, s)

        def group(g, s0):
            ids16 = idx_v[pl.ds(g * _LANES, _LANES)]
            m16 = ids16 & (_BLK - 1)
            ids_n = idx_v[pl.ds((g + 1) * _LANES, _LANES)]
            for l in range(_LANES):
                slot = ring_slot(s0, l)
                extract(m16, l, slot, g * _LANES + l)
                if l < _LANES - _NBUF:
                    fire(ids16, l + _NBUF, slot)
                else:
                    fire(ids_n, l - (_LANES - _NBUF), slot)
            s0 = s0 + (_LANES % _NBUF)
            return jnp.where(s0 >= _NBUF, s0 - _NBUF, s0)

        s_last = lax.fori_loop(0, n_groups - 1, group, jnp.int32(0))

        gl = n_groups - 1
        ids_l = idx_v[pl.ds(gl * _LANES, _LANES)]
        m_l = ids_l & (_BLK - 1)
        for l in range(_LANES):
            slot = ring_slot(s_last, l)
            extract(m_l, l, slot, gl * _LANES + l)
            if l < _LANES - _NBUF:
                fire(ids_l, l + _NBUF, slot)

        pltpu.sync_copy(acc_v, out_t.at[:, pl.ds(base, b_per_w)])


    @functools.partial(
        pl.kernel,
        out_type=jax.ShapeDtypeStruct((D, B), jnp.float32),
        mesh=mesh,
        scratch_types=[
            pltpu.VMEM((b_per_w,), jnp.int32),
            pltpu.VMEM((D, b_per_w), jnp.float32),
            pltpu.VMEM((_NBUF, D, _BLK), jnp.float32),
            pltpu.SemaphoreType.DMA((_NBUF,)),
            pltpu.SemaphoreType.DMA,
        ],
        compiler_params=pltpu.CompilerParams(needs_layout_passes=False),
    )
    def emb_kernel(tab_t, idx_hbm, pos_t, out_t,
                   idx_v, acc_v, bufs, gsem, psem):
        wid = lax.axis_index("s") * _NC + lax.axis_index("c")

        @pl.when(wid < n_active)
        def _worker():
            _worker_body(wid, tab_t, idx_hbm, pos_t, out_t,
                         idx_v, acc_v, bufs, gsem, psem)

    return emb_kernel


_K = 16
_SC_SHARE = 2048


@functools.cache
def _make_tc_kernel(N: int, D: int, V1: int, S: int):
    assert N % _K == 0
    grid = (N // _K,)

    def tmap(k, i, idx_ref):
        return (0, idx_ref[S + i * _K + k] >> 7)

    def body(idx_sref, *refs):
        blks = refs[:_K]
        pos_ref = refs[_K]
        out_ref = refs[_K + 1]
        i = pl.program_id(0)
        lanes = lax.broadcasted_iota(jnp.int32, (D, _BLK), 1)
        cols = []
        for k in range(_K):
            m = idx_sref[S + i * _K + k] & (_BLK - 1)
            sel = jnp.where(lanes == m, blks[k][...], 0.0)
            cols.append(jnp.sum(sel, axis=1, keepdims=True))
        out_ref[0] = jnp.concatenate(cols, axis=1) + pos_ref[0]

    return pl.pallas_call(
        body,
        grid_spec=pltpu.PrefetchScalarGridSpec(
            num_scalar_prefetch=1,
            grid=grid,
            in_specs=[
                *[pl.BlockSpec((D, _BLK), functools.partial(tmap, k))
                  for k in range(_K)],
                pl.BlockSpec((1, D, _K), lambda i, idx_ref: (i, 0, 0)),
            ],
            out_specs=pl.BlockSpec((1, D, _K), lambda i, idx_ref: (i, 0, 0)),
        ),
        out_shape=jax.ShapeDtypeStruct((N // _K, D, _K), jnp.float32),
    )


def kernel(x, table):
    length = x.shape[0]
    d_model = table.shape[1]
    vocab1 = table.shape[0]
    pos_np = np.ascontiguousarray(_pos_encoding_np(length, d_model).T)
    xi = x.astype(jnp.int32)
    tab_t = table.T
    s = _SC_SHARE if 0 < _SC_SHARE < length else length
    out_sc = _make_sc_kernel(s, d_model)(
        tab_t, xi, jnp.asarray(pos_np[:, :s]))
    if s < length:
        n_tc = length - s
        pos3 = np.ascontiguousarray(
            pos_np[:, s:].reshape(d_model, n_tc // _K, _K).transpose(1, 0, 2))
        out3 = _make_tc_kernel(n_tc, d_model, vocab1, s)(
            xi, *([tab_t] * _K), jnp.asarray(pos3))
        out_tc = jnp.transpose(out3, (1, 0, 2)).reshape(d_model, n_tc)
        out_t = jnp.concatenate([out_sc, out_tc], axis=1)
    else:
        out_t = out_sc
    return jnp.reshape(out_t.T, (1, length, d_model))


# speedup vs baseline: 1.8330x; 1.8330x over previous
import functools

import numpy as np
import jax
import jax.numpy as jnp
from jax import lax
from jax.experimental import pallas as pl
from jax.experimental.pallas import tpu as pltpu
from jax.experimental.pallas import tpu_sc as plsc

_NC = 2
_NS = 16
_NW = _NC * _NS
_LANES = 16
_BLK = 128
_NBUF = 8


def _pos_encoding_np(length: int, d_model: int) -> np.ndarray:
    pos = np.arange(length)[:, np.newaxis].astype(np.float32)
    i = np.arange(d_model)[np.newaxis, :].astype(np.float32)
    angle_rates = 1.0 / np.power(
        10000.0, 2.0 * np.floor(i / 2.0) / np.float32(d_model))
    a = pos * angle_rates
    a[:, 0::2] = np.sin(a[:, 0::2])
    a[:, 1::2] = np.cos(a[:, 1::2])
    return a.astype(np.float32)


@functools.cache
def _make_sc_kernel(B: int, D: int):
    assert B % _BLK == 0 and D % _LANES == 0
    b_per_w = _BLK
    n_active = B // b_per_w
    assert n_active <= _NW
    n_groups = b_per_w // _LANES
    mesh = plsc.VectorSubcoreMesh(
        core_axis_name="c", subcore_axis_name="s",
        num_cores=_NC, num_subcores=_NS)
    def _worker_body(wid, tab_t, idx_hbm, pos_t, out_t,
                     idx_v, acc_v, bufs, gsem, psem):
        base = wid * b_per_w
        pltpu.sync_copy(idx_hbm.at[pl.ds(base, b_per_w)], idx_v)
        pos_cp = pltpu.async_copy(
            pos_t.at[:, pl.ds(base, b_per_w)], acc_v, psem)

        def fire(ids16, l, slot):
            blk = ids16[l] >> 7
            off = pl.multiple_of(blk * _BLK, _BLK)
            h = D // 2
            pltpu.make_async_copy(
                tab_t.at[pl.ds(0, h), pl.ds(off, _BLK)],
                bufs.at[slot, pl.ds(0, h)], gsem.at[slot]).start()
            pltpu.make_async_copy(
                tab_t.at[pl.ds(h, h), pl.ds(off, _BLK)],
                bufs.at[slot, pl.ds(h, h)], gsem.at[slot]).start()

        def extract(m16, l, slot, col):
            pltpu.make_async_copy(
                tab_t.at[:, pl.ds(0, _BLK)],
                bufs.at[slot], gsem.at[slot]).wait()
            m = jnp.full((_LANES,), m16[l], dtype=jnp.int32)
            s = jnp.full((_LANES,), slot, dtype=jnp.int32)
            c = jnp.full((_LANES,), col, dtype=jnp.int32)
            lanes = lax.iota(jnp.int32, _LANES)
            for jc in range(D // _LANES):
                rows = lanes + jc * _LANES
                vals = plsc.load_gather(bufs, [s, rows, m])
                plsc.addupdate_scatter(acc_v, [rows, c], vals)

        ids0 = idx_v[pl.ds(0, _LANES)]
        for l in range(_NBUF):
            fire(ids0, l, l)

        pos_cp.wait()

        def ring_slot(s0, l):
            if _LANES % _NBUF == 0:
                return l % _NBUF
            s = s0 + l
            s = jnp.where(s >= _NBUF, s - _NBUF, s)
            return jnp.where(s >= _NBUF, s - _NBUF, s)

        def group(g, s0):
            ids16 = idx_v[pl.ds(g * _LANES, _LANES)]
            m16 = ids16 & (_BLK - 1)
            ids_n = idx_v[pl.ds((g + 1) * _LANES, _LANES)]
            for l in range(_LANES):
                slot = ring_slot(s0, l)
                extract(m16, l, slot, g * _LANES + l)
                if l < _LANES - _NBUF:
                    fire(ids16, l + _NBUF, slot)
                else:
                    fire(ids_n, l - (_LANES - _NBUF), slot)
            s0 = s0 + (_LANES % _NBUF)
            return jnp.where(s0 >= _NBUF, s0 - _NBUF, s0)

        s_last = lax.fori_loop(0, n_groups - 1, group, jnp.int32(0))

        gl = n_groups - 1
        ids_l = idx_v[pl.ds(gl * _LANES, _LANES)]
        m_l = ids_l & (_BLK - 1)
        for l in range(_LANES):
            slot = ring_slot(s_last, l)
            extract(m_l, l, slot, gl * _LANES + l)
            if l < _LANES - _NBUF:
                fire(ids_l, l + _NBUF, slot)

        pltpu.sync_copy(acc_v, out_t.at[:, pl.ds(base, b_per_w)])


    @functools.partial(
        pl.kernel,
        out_type=jax.ShapeDtypeStruct((D, B), jnp.float32),
        mesh=mesh,
        scratch_types=[
            pltpu.VMEM((b_per_w,), jnp.int32),
            pltpu.VMEM((D, b_per_w), jnp.float32),
            pltpu.VMEM((_NBUF, D, _BLK), jnp.float32),
            pltpu.SemaphoreType.DMA((_NBUF,)),
            pltpu.SemaphoreType.DMA,
        ],
        compiler_params=pltpu.CompilerParams(needs_layout_passes=False),
    )
    def emb_kernel(tab_t, idx_hbm, pos_t, out_t,
                   idx_v, acc_v, bufs, gsem, psem):
        wid = lax.axis_index("s") * _NC + lax.axis_index("c")

        @pl.when(wid < n_active)
        def _worker():
            _worker_body(wid, tab_t, idx_hbm, pos_t, out_t,
                         idx_v, acc_v, bufs, gsem, psem)

    return emb_kernel


_K = 16
_SC_SHARE = 4096


@functools.cache
def _make_tc_kernel(N: int, D: int, V1: int, S: int):
    assert N % _K == 0
    grid = (N // _K,)

    def tmap(k, i, idx_ref):
        return (0, idx_ref[S + i * _K + k] >> 7)

    def body(idx_sref, *refs):
        blks = refs[:_K]
        pos_ref = refs[_K]
        out_ref = refs[_K + 1]
        i = pl.program_id(0)
        lanes = lax.broadcasted_iota(jnp.int32, (D, _BLK), 1)
        cols = []
        for k in range(_K):
            m = idx_sref[S + i * _K + k] & (_BLK - 1)
            sel = jnp.where(lanes == m, blks[k][...], 0.0)
            cols.append(jnp.sum(sel, axis=1, keepdims=True))
        out_ref[0] = jnp.concatenate(cols, axis=1) + pos_ref[0]

    return pl.pallas_call(
        body,
        grid_spec=pltpu.PrefetchScalarGridSpec(
            num_scalar_prefetch=1,
            grid=grid,
            in_specs=[
                *[pl.BlockSpec((D, _BLK), functools.partial(tmap, k))
                  for k in range(_K)],
                pl.BlockSpec((1, D, _K), lambda i, idx_ref: (i, 0, 0)),
            ],
            out_specs=pl.BlockSpec((1, D, _K), lambda i, idx_ref: (i, 0, 0)),
        ),
        out_shape=jax.ShapeDtypeStruct((N // _K, D, _K), jnp.float32),
    )


def kernel(x, table):
    length = x.shape[0]
    d_model = table.shape[1]
    vocab1 = table.shape[0]
    pos_np = np.ascontiguousarray(_pos_encoding_np(length, d_model).T)
    xi = x.astype(jnp.int32)
    tab_t = table.T
    s = _SC_SHARE if 0 < _SC_SHARE < length else length
    out_sc = _make_sc_kernel(s, d_model)(
        tab_t, xi, jnp.asarray(pos_np[:, :s]))
    if s < length:
        n_tc = length - s
        pos3 = np.ascontiguousarray(
            pos_np[:, s:].reshape(d_model, n_tc // _K, _K).transpose(1, 0, 2))
        out3 = _make_tc_kernel(n_tc, d_model, vocab1, s)(
            xi, *([tab_t] * _K), jnp.asarray(pos3))
        out_tc = jnp.transpose(out3, (1, 0, 2)).reshape(d_model, n_tc)
        out_t = jnp.concatenate([out_sc, out_tc], axis=1)
    else:
        out_t = out_sc
    return jnp.reshape(out_t.T, (1, length, d_model))
